# baseline (speedup 1.0000x reference)
import functools

import jax
import jax.numpy as jnp
import numpy as np
from jax import lax
from jax.experimental import pallas as pl
from jax.experimental.pallas import tpu as pltpu
from jax.experimental.pallas import tpu_sc as plsc

_SEQ_LEN = 8192
_EMBED_DIM = 64


def _pos_encoding_np(position, d_model):
    i = np.arange(d_model)[np.newaxis, :]
    pos = np.arange(position)[:, np.newaxis]
    angle_rates = 1.0 / np.power(10000, 2 * (i // 2) / np.float32(d_model))
    angle_rads = pos * angle_rates
    angle_rads[:, 0::2] = np.sin(angle_rads[:, 0::2])
    angle_rads[:, 1::2] = np.cos(angle_rads[:, 1::2])
    return angle_rads.astype(np.float32)


_POS = _pos_encoding_np(_SEQ_LEN, _EMBED_DIM)

_INFO = plsc.get_sparse_core_info()
_NC, _NS = _INFO.num_cores, _INFO.num_subcores
_NW = _NC * _NS
_B_PER_W = _SEQ_LEN // _NW


def _sc_body(x_hbm, pos_hbm, table_hbm, out_hbm, idx_v, rows_v, sem_g, sem_p):
    wid = lax.axis_index("s") * _NC + lax.axis_index("c")
    base = wid * _B_PER_W
    pos_cp = pltpu.async_copy(pos_hbm.at[pl.ds(base, _B_PER_W)], rows_v, sem_p)
    pltpu.sync_copy(x_hbm.at[pl.ds(base, _B_PER_W)], idx_v)
    pos_cp.wait()
    pltpu.async_copy(table_hbm.at[idx_v], rows_v, sem_g, add=True).wait()
    pltpu.sync_copy(rows_v, out_hbm.at[pl.ds(base, _B_PER_W)])


@jax.jit
def _embed(x_i32, pos, table):
    mesh = plsc.VectorSubcoreMesh(core_axis_name="c", subcore_axis_name="s")
    return pl.kernel(
        _sc_body,
        out_type=jax.ShapeDtypeStruct((_SEQ_LEN, _EMBED_DIM), jnp.float32),
        mesh=mesh,
        scratch_types=[
            pltpu.VMEM((_B_PER_W,), jnp.int32),
            pltpu.VMEM((_B_PER_W, _EMBED_DIM), jnp.float32),
            pltpu.SemaphoreType.DMA,
            pltpu.SemaphoreType.DMA,
        ],
        compiler_params=pltpu.CompilerParams(use_tc_tiling_on_sc=False),
    )(x_i32, pos, table)


def kernel(x, table):
    x_i32 = x.astype(jnp.int32)
    pos = jnp.asarray(_POS)
    out = _embed(x_i32, pos, table)
    return out.reshape(1, _SEQ_LEN, _EMBED_DIM)


# speedup vs baseline: 1.3098x; 1.3098x over previous
import functools

import jax
import jax.numpy as jnp
import numpy as np
from jax import lax
from jax.experimental import pallas as pl
from jax.experimental.pallas import tpu as pltpu
from jax.experimental.pallas import tpu_sc as plsc

_SEQ_LEN = 8192
_EMBED_DIM = 64


def _pos_encoding_np(position, d_model):
    i = np.arange(d_model)[np.newaxis, :]
    pos = np.arange(position)[:, np.newaxis]
    angle_rates = 1.0 / np.power(10000, 2 * (i // 2) / np.float32(d_model))
    angle_rads = pos * angle_rates
    angle_rads[:, 0::2] = np.sin(angle_rads[:, 0::2])
    angle_rads[:, 1::2] = np.cos(angle_rads[:, 1::2])
    return angle_rads.astype(np.float32)


_POS = _pos_encoding_np(_SEQ_LEN, _EMBED_DIM)

_INFO = plsc.get_sparse_core_info()
_NC, _NS = _INFO.num_cores, _INFO.num_subcores
_NW = _NC * _NS
_B_PER_W = _SEQ_LEN // _NW
_VPR = _EMBED_DIM // 16


def _sc_body(x_hbm, pos_hbm, table_hbm, out_hbm, idx_v, rows_v, pos_v,
             sem_g, sem_p):
    wid = lax.axis_index("s") * _NC + lax.axis_index("c")
    base = wid * _B_PER_W
    pos_cp = pltpu.async_copy(pos_hbm.at[pl.ds(base, _B_PER_W)], pos_v, sem_p)
    pltpu.sync_copy(x_hbm.at[pl.ds(base, _B_PER_W)], idx_v)

    def issue(g, _):
        v = idx_v[pl.ds(g * 16, 16)]
        for j in range(16):
            pltpu.async_copy(table_hbm.at[pl.ds(v[j], 1)],
                             rows_v.at[pl.ds(g * 16 + j, 1)], sem_g)
        return 0

    lax.fori_loop(0, _B_PER_W // 16, issue, 0)

    def drain(i, _):
        pltpu.make_async_copy(table_hbm.at[pl.ds(0, 1)],
                              rows_v.at[pl.ds(i, 1)], sem_g).wait()
        return 0

    lax.fori_loop(0, _B_PER_W, drain, 0, unroll=4)
    pos_cp.wait()

    def add(i, _):
        r = i // _VPR
        c = (i % _VPR) * 16
        rows_v[r, pl.ds(c, 16)] = rows_v[r, pl.ds(c, 16)] + pos_v[r, pl.ds(c, 16)]
        return 0

    lax.fori_loop(0, _B_PER_W * _VPR, add, 0, unroll=8)

    pltpu.sync_copy(rows_v, out_hbm.at[pl.ds(base, _B_PER_W)])


def _embed(x_i32, pos, table):
    mesh = plsc.VectorSubcoreMesh(core_axis_name="c", subcore_axis_name="s")
    return pl.kernel(
        _sc_body,
        out_type=jax.ShapeDtypeStruct((_SEQ_LEN, _EMBED_DIM), jnp.float32),
        mesh=mesh,
        scratch_types=[
            pltpu.VMEM((_B_PER_W,), jnp.int32),
            pltpu.VMEM((_B_PER_W, _EMBED_DIM), jnp.float32),
            pltpu.VMEM((_B_PER_W, _EMBED_DIM), jnp.float32),
            pltpu.SemaphoreType.DMA,
            pltpu.SemaphoreType.DMA,
        ],
        compiler_params=pltpu.CompilerParams(use_tc_tiling_on_sc=True),
    )(x_i32, pos, table)


def kernel(x, table):
    x_i32 = x.astype(jnp.int32)
    pos = jnp.asarray(_POS)
    out = _embed(x_i32, pos, table)
    return out.reshape(1, _SEQ_LEN, _EMBED_DIM)


# speedup vs baseline: 1.3111x; 1.0010x over previous
import functools

import jax
import jax.numpy as jnp
import numpy as np
from jax import lax
from jax.experimental import pallas as pl
from jax.experimental.pallas import tpu as pltpu
from jax.experimental.pallas import tpu_sc as plsc

_SEQ_LEN = 8192
_EMBED_DIM = 64


def _pos_encoding_np(position, d_model):
    i = np.arange(d_model)[np.newaxis, :]
    pos = np.arange(position)[:, np.newaxis]
    angle_rates = 1.0 / np.power(10000, 2 * (i // 2) / np.float32(d_model))
    angle_rads = pos * angle_rates
    angle_rads[:, 0::2] = np.sin(angle_rads[:, 0::2])
    angle_rads[:, 1::2] = np.cos(angle_rads[:, 1::2])
    return angle_rads.astype(np.float32)


_POS = _pos_encoding_np(_SEQ_LEN, _EMBED_DIM)

_INFO = plsc.get_sparse_core_info()
_NC, _NS = _INFO.num_cores, _INFO.num_subcores
_NW = _NC * _NS
_B_PER_W = _SEQ_LEN // _NW
_VPR = _EMBED_DIM // 16


def _sc_body(x_hbm, pos_hbm, table_hbm, out_hbm, idx_v, rows_v, pos_v,
             sem_g, sem_p):
    wid = lax.axis_index("s") * _NC + lax.axis_index("c")
    base = wid * _B_PER_W
    with jax.named_scope("stage"):
        pos_cp = pltpu.async_copy(pos_hbm.at[pl.ds(base, _B_PER_W)], pos_v,
                                  sem_p)
        pltpu.sync_copy(x_hbm.at[pl.ds(base, _B_PER_W)], idx_v)

    with jax.named_scope("issue"):
        def issue(g, _):
            v = idx_v[pl.ds(g * 16, 16)]
            for j in range(16):
                pltpu.async_copy(table_hbm.at[pl.ds(v[j], 1)],
                                 rows_v.at[pl.ds(g * 16 + j, 1)], sem_g)
            return 0

        lax.fori_loop(0, _B_PER_W // 16, issue, 0)

    with jax.named_scope("drain"):
        def drain(i, _):
            pltpu.make_async_copy(table_hbm.at[pl.ds(0, 1)],
                                  rows_v.at[pl.ds(i, 1)], sem_g).wait()
            return 0

        lax.fori_loop(0, _B_PER_W, drain, 0, unroll=4)
        pos_cp.wait()

    with jax.named_scope("add"):
        def add(i, _):
            r = i // _VPR
            c = (i % _VPR) * 16
            rows_v[r, pl.ds(c, 16)] = (rows_v[r, pl.ds(c, 16)]
                                       + pos_v[r, pl.ds(c, 16)])
            return 0

        lax.fori_loop(0, _B_PER_W * _VPR, add, 0, unroll=8)

    with jax.named_scope("writeback"):
        pltpu.sync_copy(rows_v, out_hbm.at[pl.ds(base, _B_PER_W)])


def _embed(x_i32, pos, table):
    mesh = plsc.VectorSubcoreMesh(core_axis_name="c", subcore_axis_name="s")
    return pl.kernel(
        _sc_body,
        out_type=jax.ShapeDtypeStruct((_SEQ_LEN, _EMBED_DIM), jnp.float32),
        mesh=mesh,
        scratch_types=[
            pltpu.VMEM((_B_PER_W,), jnp.int32),
            pltpu.VMEM((_B_PER_W, _EMBED_DIM), jnp.float32),
            pltpu.VMEM((_B_PER_W, _EMBED_DIM), jnp.float32),
            pltpu.SemaphoreType.DMA,
            pltpu.SemaphoreType.DMA,
        ],
        compiler_params=pltpu.CompilerParams(use_tc_tiling_on_sc=True),
    )(x_i32, pos, table)


def kernel(x, table):
    x_i32 = x.astype(jnp.int32)
    pos = jnp.asarray(_POS)
    out = _embed(x_i32, pos, table)
    return out.reshape(1, _SEQ_LEN, _EMBED_DIM)
